# Initial kernel scaffold; baseline (speedup 1.0000x reference)
import jax
import jax.numpy as jnp
from jax.experimental import pallas as pl


def kernel(bond_angle, params, atom_x, edge_index, edge_attr_b, batch, bond_edge_index):
    raise NotImplementedError("write your pallas kernel here")


# speedup vs baseline: 1.5818x; 1.5818x over previous
import functools

import jax
import jax.numpy as jnp
from jax import lax
from jax.experimental import pallas as pl
from jax.experimental.pallas import tpu as pltpu
from jax.experimental.pallas import tpu_sc as plsc

G = 256
CH = 512
RB = 512

_SEQ = pltpu.CompilerParams(dimension_semantics=("arbitrary",))


def _cdiv(a, b):
    return (a + b - 1) // b


def _pad_rows(x, n, val=0):
    if x.shape[0] == n:
        return x
    pad = [(0, n - x.shape[0])] + [(0, 0)] * (x.ndim - 1)
    return jnp.pad(x, pad, constant_values=val)


def _worklist(dst_sorted_padded, nchunks, ch, nblocks, nb):
    bounds = jnp.arange(nblocks + 1, dtype=jnp.int32) * nb
    pos = jnp.searchsorted(dst_sorted_padded, bounds, side="left").astype(jnp.int32)
    s, e = pos[:-1], pos[1:]
    c_lo = s // ch
    c_hi = jnp.maximum((e - 1) // ch, c_lo)
    counts = c_hi - c_lo + 1
    nw = nblocks + nchunks - 1
    offs = jnp.concatenate(
        [jnp.zeros((1,), jnp.int32), jnp.cumsum(counts)[:-1].astype(jnp.int32)]
    )
    k = offs[-1] + counts[-1]
    ar = jnp.arange(nw, dtype=jnp.int32)
    blk = jnp.clip(
        jnp.searchsorted(offs, ar, side="right").astype(jnp.int32) - 1, 0, nblocks - 1
    )
    chk = jnp.clip(c_lo[blk] + (ar - offs[blk]), 0, nchunks - 1)
    first = ((ar == offs[blk]) & (ar < k)).astype(jnp.int32)
    act = (ar < k).astype(jnp.int32)
    return chk, blk, first, act


def _split_dot(oh, b):
    bh = b.astype(jnp.bfloat16)
    bl = (b - bh.astype(jnp.float32)).astype(jnp.bfloat16)
    o = oh.astype(jnp.bfloat16)
    return (jnp.dot(o, bh, preferred_element_type=jnp.float32)
            + jnp.dot(o, bl, preferred_element_type=jnp.float32))


def _multi_onehot(cid, dims, ncols):
    r = cid
    cols = []
    off = 0
    strides = []
    st = 1
    for d in reversed(dims):
        strides.append(st)
        st *= d
    strides = list(reversed(strides))
    for d, s in zip(dims, strides):
        a = (r // s) % d
        cols.append(off + a)
        off += d
    it = lax.broadcasted_iota(jnp.int32, (cid.shape[0], ncols), 1)
    oh = it < 0
    for c in cols:
        oh = oh | (it == c[:, None])
    return oh.astype(jnp.float32)


def _scatter_tail(blk_r, first_r, act_r, dst_r, msg, out_r, i, nb):
    dst = dst_r[0, 0, :]
    local = dst - blk_r[i] * nb
    rowi = lax.broadcasted_iota(jnp.int32, (nb, dst.shape[0]), 0)
    mask = (local[None, :] == rowi) & (act_r[i] == 1)
    oh = mask.astype(jnp.float32)

    @pl.when(first_r[i] == 1)
    def _():
        out_r[...] = jnp.zeros_like(out_r)

    out_r[...] += _split_dot(oh, msg)


def _embed_kernel(cid_r, tab_r, out_r, *, dims, ncols):
    cid = cid_r[0, 0, :]
    oh = _multi_onehot(cid, dims, ncols)
    out_r[...] = _split_dot(oh, tab_r[...])


def _embed(cid3d, tab, dims, nrows, d):
    nchunks = cid3d.shape[0]
    ncols = tab.shape[0]
    return pl.pallas_call(
        functools.partial(_embed_kernel, dims=dims, ncols=ncols),
        grid=(nchunks,),
        in_specs=[
            pl.BlockSpec((1, 1, cid3d.shape[2]), lambda i: (i, 0, 0)),
            pl.BlockSpec((ncols, d), lambda i: (0, 0)),
        ],
        out_specs=pl.BlockSpec((cid3d.shape[2], d), lambda i: (i, 0)),
        out_shape=jax.ShapeDtypeStruct((nrows, d), jnp.float32),
        compiler_params=_SEQ,
    )(cid3d, tab)


def _atom_scatter_kernel(chk_r, blk_r, first_r, act_r, gx_r, eh_r, dst_r, w_r, b_r,
                         out_r, *, nb):
    i = pl.program_id(0)
    ea = jnp.dot(eh_r[...], w_r[...], preferred_element_type=jnp.float32) + b_r[...]
    msg = jnp.maximum(gx_r[...] + ea, 0.0)
    _scatter_tail(blk_r, first_r, act_r, dst_r, msg, out_r, i, nb)


def _bond_scatter_kernel(chk_r, blk_r, first_r, act_r, cid_r, ba_r, tab_r, w1_r,
                         b1_r, wp_r, bp_r, dst_r, out_r, *, nb, dims, ncols):
    i = pl.program_id(0)
    cid = cid_r[0, 0, :]
    oh = _multi_onehot(cid, dims, ncols)
    cea = _split_dot(oh, tab_r[...])
    ba = ba_r[0, 0, :]
    z = jnp.maximum(ba[:, None] * w1_r[...] + b1_r[...], 0.0)
    ea = jnp.dot(z, wp_r[...], preferred_element_type=jnp.float32) + bp_r[...]
    msg = jnp.maximum(cea + ea, 0.0)
    _scatter_tail(blk_r, first_r, act_r, dst_r, msg, out_r, i, nb)


def _mlp_core(h, w1_r, b1_r, w2_r, b2_r, g_r, be_r):
    z = jnp.maximum(jnp.dot(h, w1_r[...], preferred_element_type=jnp.float32)
                    + b1_r[...], 0.0)
    y = jnp.dot(z, w2_r[...], preferred_element_type=jnp.float32) + b2_r[...]
    m = jnp.mean(y, axis=1, keepdims=True)
    v = jnp.mean((y - m) ** 2, axis=1, keepdims=True)
    return g_r[...] * (y - m) * lax.rsqrt(v + 1e-5) + be_r[...]


def _stats_tail(t, st_r, i, nreal, rb):
    row = lax.broadcasted_iota(jnp.int32, (rb, 1), 0) + i * rb
    tm = jnp.where(row < nreal, t, 0.0)

    @pl.when(i == 0)
    def _():
        st_r[...] = jnp.zeros_like(st_r)

    st_r[0:1, :] += jnp.sum(tm, axis=0, keepdims=True)
    st_r[1:2, :] += jnp.sum(tm * tm, axis=0, keepdims=True)


def _atom_mlp_kernel(x_r, agg_r, w1_r, b1_r, w2_r, b2_r, g_r, be_r, t_r, st_r,
                     *, nreal, rb):
    i = pl.program_id(0)
    t = _mlp_core(x_r[...] + agg_r[...], w1_r, b1_r, w2_r, b2_r, g_r, be_r)
    t_r[...] = t
    _stats_tail(t, st_r, i, nreal, rb)


def _bond_mlp_kernel(cid_r, tab_r, agg_r, w1_r, b1_r, w2_r, b2_r, g_r, be_r,
                     t_r, st_r, *, nreal, rb, dims, ncols):
    i = pl.program_id(0)
    cid = cid_r[0, 0, :]
    oh = _multi_onehot(cid, dims, ncols)
    cea = _split_dot(oh, tab_r[...])
    t = _mlp_core(cea + agg_r[...], w1_r, b1_r, w2_r, b2_r, g_r, be_r)
    t_r[...] = t
    _stats_tail(t, st_r, i, nreal, rb)


def _affine_kernel(st_r, t_r, res_r, w_r, b_r, al_r, out_r, *, nreal, relu):
    al = al_r[...]
    m = st_r[0:1, :] / nreal
    v = st_r[1:2, :] / nreal - (2.0 * al - al * al) * m * m
    y = w_r[...] * (t_r[...] - al * m) * lax.rsqrt(v + 1e-5) + b_r[...]
    if relu:
        y = jnp.maximum(y, 0.0)
    out_r[...] = y + res_r[...]


def _pool_kernel(x_r, b_r, out_r, sum_s, cnt_s, *, nchunks, d):
    i = pl.program_id(0)

    @pl.when(i == 0)
    def _():
        sum_s[...] = jnp.zeros_like(sum_s)
        cnt_s[...] = jnp.zeros_like(cnt_s)

    b = b_r[0, 0, :]
    rowi = lax.broadcasted_iota(jnp.int32, (G, b.shape[0]), 0)
    oh = (b[None, :] == rowi).astype(jnp.float32)
    sum_s[...] += _split_dot(oh, x_r[...])
    cnt_s[...] = cnt_s[...] + jnp.sum(oh, axis=1, keepdims=True)

    @pl.when(i == nchunks - 1)
    def _():
        out_r[...] = sum_s[...] / jnp.maximum(cnt_s[...][:, 0:1], 1.0)


def _sc_gather(table, idx, b):
    d = table.shape[1]
    info = plsc.get_sparse_core_info()
    nwk = info.num_cores * info.num_subcores
    per = b // nwk
    c = 128
    n_iter = per // c
    mesh = plsc.VectorSubcoreMesh(core_axis_name="c", subcore_axis_name="s")

    @functools.partial(
        pl.kernel,
        mesh=mesh,
        out_type=jax.ShapeDtypeStruct((b, d), jnp.float32),
        scratch_types=[
            pltpu.VMEM((c,), jnp.int32),
            pltpu.VMEM((c, d), jnp.float32),
            pltpu.SemaphoreType.DMA,
        ],
    )
    def k(table_hbm, idx_hbm, out_hbm, idx_v, rows_v, sem):
        wid = lax.axis_index("s") * info.num_cores + lax.axis_index("c")
        base = wid * per

        def body(j, carry):
            off = base + j * c
            pltpu.sync_copy(idx_hbm.at[pl.ds(off, c)], idx_v)
            pltpu.async_copy(table_hbm.at[idx_v], rows_v, sem).wait()
            pltpu.sync_copy(rows_v, out_hbm.at[pl.ds(off, c)])
            return carry

        lax.fori_loop(0, n_iter, body, 0)

    return k(table, idx)


def _gather_rows(table, idx, b):
    return _sc_gather(table, idx, b)


def kernel(bond_angle, params, atom_x, edge_index, edge_attr_b, batch,
           bond_edge_index):
    n = atom_x.shape[0]
    e = edge_index.shape[1]
    e2 = bond_edge_index.shape[1]
    d = params["atom_tables"][0].shape[1]
    nlayers = len(params["layers"])

    nb_a = 256
    nblk_a = _cdiv(n, nb_a)
    npad = nblk_a * nb_a
    nchk_e = _cdiv(e, CH)
    epad = nchk_e * CH
    nb_b = 512
    nblk_b = epad // nb_b
    nchk_e2 = _cdiv(e2, CH)
    e2pad = nchk_e2 * CH
    bg = _cdiv(epad, 32 * 128) * (32 * 128)

    src = edge_index[0].astype(jnp.int32)
    dst = edge_index[1].astype(jnp.int32)
    perm = jnp.argsort(dst)
    src_s = src[perm]
    dst_s = dst[perm]
    inv = jnp.zeros((e,), jnp.int32).at[perm].set(jnp.arange(e, dtype=jnp.int32))
    bsrc = inv[bond_edge_index[0].astype(jnp.int32)]
    bdst = inv[bond_edge_index[1].astype(jnp.int32)]
    perm2 = jnp.argsort(bdst)
    bsrc_s = bsrc[perm2]
    bdst_s = bdst[perm2]
    ba_s = bond_angle[:, 0][perm2]

    bdims = tuple(t.shape[0] for t in params["bond_init_tables"])
    adims = tuple(t.shape[0] for t in params["atom_tables"])
    ea3 = edge_attr_b.astype(jnp.int32)
    cattr = (ea3[:, 0] * (bdims[1] * bdims[2]) + ea3[:, 1] * bdims[2] + ea3[:, 2])[perm]
    cattr2 = cattr[bsrc_s]
    astr = []
    st = 1
    for dd in reversed(adims):
        astr.append(st)
        st *= dd
    astr = list(reversed(astr))
    ax = atom_x.astype(jnp.int32)
    catom = sum(ax[:, k] * astr[k] for k in range(len(adims)))

    src_g = _pad_rows(src_s, bg, 0)
    dst_p = _pad_rows(dst_s, epad, npad)
    dst3d = dst_p.reshape(nchk_e, 1, CH)
    cattr_p = _pad_rows(cattr, epad, 0)
    cattr3d = cattr_p.reshape(nchk_e, 1, CH)
    bdst_p = _pad_rows(bdst_s, e2pad, epad)
    bdst3d = bdst_p.reshape(nchk_e2, 1, CH)
    cattr2_p = _pad_rows(cattr2, e2pad, 0)
    cattr23d = cattr2_p.reshape(nchk_e2, 1, CH)
    ba_p = _pad_rows(ba_s, e2pad, 0.0)
    ba3d = ba_p.reshape(nchk_e2, 1, CH)
    catom_p = _pad_rows(catom, npad, 0)
    catom3d = catom_p.reshape(npad // CH, 1, CH)
    batch_p = _pad_rows(batch.astype(jnp.int32), npad, G)
    batch3d = batch_p.reshape(npad // CH, 1, CH)

    wl_a = _worklist(dst_p, nchk_e, CH, nblk_a, nb_a)
    wl_b = _worklist(bdst_p, nchk_e2, CH, nblk_b, nb_b)

    def cat_pad(tabs, ncols):
        t = jnp.concatenate(tabs, axis=0)
        return _pad_rows(t, ncols, 0.0)

    nca = _cdiv(sum(adims), 8) * 8
    ncb = _cdiv(sum(bdims), 8) * 8
    tatom = cat_pad(params["atom_tables"], nca)
    tbond0 = cat_pad(params["bond_init_tables"], ncb)
    row = lambda v: v.reshape(1, -1)

    x = _embed(catom3d, tatom, adims, npad, d)
    eh = _embed(cattr3d, tbond0, bdims, epad, d)

    def scatter_call(kern, nblk, nb, wl, ins, specs):
        nw = wl[0].shape[0]
        gs = pltpu.PrefetchScalarGridSpec(
            num_scalar_prefetch=4,
            grid=(nw,),
            in_specs=specs,
            out_specs=pl.BlockSpec(
                (nb, d), lambda i, c, b_, f, a: (b_[i], 0)),
        )
        return pl.pallas_call(
            kern, grid_spec=gs,
            out_shape=jax.ShapeDtypeStruct((nblk * nb, d), jnp.float32),
            compiler_params=_SEQ,
        )(*wl, *ins)

    def mlp_call(kern, ins, specs, nrows, nreal):
        nchunks = nrows // RB
        return pl.pallas_call(
            functools.partial(kern, nreal=nreal, rb=RB),
            grid=(nchunks,),
            in_specs=specs,
            out_specs=[
                pl.BlockSpec((RB, d), lambda i: (i, 0)),
                pl.BlockSpec((8, d), lambda i: (0, 0)),
            ],
            out_shape=[
                jax.ShapeDtypeStruct((nrows, d), jnp.float32),
                jax.ShapeDtypeStruct((8, d), jnp.float32),
            ],
            compiler_params=_SEQ,
        )(*ins)

    def affine_call(st, t, res, gn, nreal, relu):
        nrows = t.shape[0]
        return pl.pallas_call(
            functools.partial(_affine_kernel, nreal=float(nreal), relu=relu),
            grid=(nrows // RB,),
            in_specs=[
                pl.BlockSpec((8, d), lambda i: (0, 0)),
                pl.BlockSpec((RB, d), lambda i: (i, 0)),
                pl.BlockSpec((RB, d), lambda i: (i, 0)),
                pl.BlockSpec((1, d), lambda i: (0, 0)),
                pl.BlockSpec((1, d), lambda i: (0, 0)),
                pl.BlockSpec((1, d), lambda i: (0, 0)),
            ],
            out_specs=pl.BlockSpec((RB, d), lambda i: (i, 0)),
            out_shape=jax.ShapeDtypeStruct((nrows, d), jnp.float32),
            compiler_params=_SEQ,
        )(st, t, res, row(gn["w"]), row(gn["b"]), row(gn["alpha"]))

    cmap = lambda i, c, b_, f, a: (c[i], 0)
    cmap3 = lambda i, c, b_, f, a: (c[i], 0, 0)
    fmap = lambda i, c, b_, f, a: (0, 0)

    for li in range(nlayers):
        lp = params["layers"][li]
        last = li == nlayers - 1

        gx = _gather_rows(x, src_g, bg)
        ac = lp["atom_conv"]
        agg = scatter_call(
            functools.partial(_atom_scatter_kernel, nb=nb_a),
            nblk_a, nb_a, wl_a,
            (gx, eh, dst3d, ac["lin_w"], row(ac["lin_b"])),
            [
                pl.BlockSpec((CH, d), cmap),
                pl.BlockSpec((CH, d), cmap),
                pl.BlockSpec((1, 1, CH), cmap3),
                pl.BlockSpec((d, d), fmap),
                pl.BlockSpec((1, d), fmap),
            ])
        t, stt = mlp_call(
            functools.partial(_atom_mlp_kernel),
            (x, agg, ac["w1"], row(ac["b1"]), ac["w2"], row(ac["b2"]),
             row(lp["ln_atom"]["g"]), row(lp["ln_atom"]["b"])),
            [
                pl.BlockSpec((RB, d), lambda i: (i, 0)),
                pl.BlockSpec((RB, d), lambda i: (i, 0)),
                pl.BlockSpec((d, 2 * d), lambda i: (0, 0)),
                pl.BlockSpec((1, 2 * d), lambda i: (0, 0)),
                pl.BlockSpec((2 * d, d), lambda i: (0, 0)),
                pl.BlockSpec((1, d), lambda i: (0, 0)),
                pl.BlockSpec((1, d), lambda i: (0, 0)),
                pl.BlockSpec((1, d), lambda i: (0, 0)),
            ], npad, n)
        x = affine_call(stt, t, x, lp["gn_atom"], n, last)

        if not last:
            bc = lp["bond_conv"]
            ang = lp["angle"]
            tbond = cat_pad(lp["bond_tables"], ncb)
            wp = ang["w2"] @ bc["lin_w"]
            bp = ang["b2"] @ bc["lin_w"] + bc["lin_b"]
            agg2 = scatter_call(
                functools.partial(_bond_scatter_kernel, nb=nb_b, dims=bdims,
                                  ncols=ncb),
                nblk_b, nb_b, wl_b,
                (cattr23d, ba3d, tbond, ang["w1"], row(ang["b1"]), wp, row(bp),
                 bdst3d),
                [
                    pl.BlockSpec((1, 1, CH), cmap3),
                    pl.BlockSpec((1, 1, CH), cmap3),
                    pl.BlockSpec((ncb, d), fmap),
                    pl.BlockSpec((1, d), fmap),
                    pl.BlockSpec((1, d), fmap),
                    pl.BlockSpec((d, d), fmap),
                    pl.BlockSpec((1, d), fmap),
                    pl.BlockSpec((1, 1, CH), cmap3),
                ])
            t2, st2 = mlp_call(
                functools.partial(_bond_mlp_kernel, dims=bdims, ncols=ncb),
                (cattr3d, tbond, agg2, bc["w1"], row(bc["b1"]), bc["w2"],
                 row(bc["b2"]), row(lp["ln_bond"]["g"]), row(lp["ln_bond"]["b"])),
                [
                    pl.BlockSpec((1, 1, RB), lambda i: (i, 0, 0)),
                    pl.BlockSpec((ncb, d), lambda i: (0, 0)),
                    pl.BlockSpec((RB, d), lambda i: (i, 0)),
                    pl.BlockSpec((d, 2 * d), lambda i: (0, 0)),
                    pl.BlockSpec((1, 2 * d), lambda i: (0, 0)),
                    pl.BlockSpec((2 * d, d), lambda i: (0, 0)),
                    pl.BlockSpec((1, d), lambda i: (0, 0)),
                    pl.BlockSpec((1, d), lambda i: (0, 0)),
                    pl.BlockSpec((1, d), lambda i: (0, 0)),
                ], epad, e)
            eh = affine_call(st2, t2, eh, lp["gn_bond"], e, False)

    nch_p = npad // CH
    out = pl.pallas_call(
        functools.partial(_pool_kernel, nchunks=nch_p, d=d),
        grid=(nch_p,),
        in_specs=[
            pl.BlockSpec((CH, d), lambda i: (i, 0)),
            pl.BlockSpec((1, 1, CH), lambda i: (i, 0, 0)),
        ],
        out_specs=pl.BlockSpec((G, d), lambda i: (0, 0)),
        out_shape=jax.ShapeDtypeStruct((G, d), jnp.float32),
        scratch_shapes=[
            pltpu.VMEM((G, d), jnp.float32),
            pltpu.VMEM((G, 128), jnp.float32),
        ],
        compiler_params=_SEQ,
    )(x, batch3d)
    return out
